# baseline (speedup 1.0000x reference)
import jax
import jax.numpy as jnp
from jax import lax
from jax.experimental import pallas as pl
from jax.experimental.pallas import tpu as pltpu

N, S, K = 4096, 200, 100
IDS_PER_ROW = 32
W = IDS_PER_ROW * K
ROWS = N * S // IDS_PER_ROW
R_BLK = 256
GRID = ROWS // R_BLK
NBUF = 6
LT = W // 128


def _body(in_ref, out_hbm, buf, sems):
    i = pl.program_id(0)
    slot = lax.rem(i, NBUF)

    @pl.when(i >= NBUF)
    def _wait_prev():
        pltpu.make_async_copy(
            buf.at[slot],
            out_hbm.at[pl.ds((i - NBUF) * R_BLK, R_BLK)],
            sems.at[slot],
        ).wait()

    ids_blk = in_ref[...]
    lane = lax.broadcasted_iota(jnp.int32, (R_BLK, 128), 1)

    col_bcast = {}

    def col(c):
        if c not in col_bcast:
            col_bcast[c] = jnp.broadcast_to(ids_blk[:, c:c + 1], (R_BLK, 128))
        return col_bcast[c]

    parts = []
    for t in range(LT):
        j0 = 128 * t
        c0 = j0 // K
        split0 = K - (j0 % K)
        a = col(c0)
        if split0 < 128:
            nxt = col(c0 + 1)
            if split0 + K < 128:
                nxt = jnp.where(lane < split0 + K, nxt, col(c0 + 2))
            a = jnp.where(lane < split0, a, nxt)
        k_pat = (lane + (j0 % K)) % K
        parts.append((a == k_pat).astype(jnp.float32))

    chunk = jnp.concatenate(parts, axis=1)
    buf[pl.ds(slot, 1)] = chunk.reshape(1, R_BLK, W)

    pltpu.make_async_copy(
        buf.at[slot],
        out_hbm.at[pl.ds(i * R_BLK, R_BLK)],
        sems.at[slot],
    ).start()

    @pl.when(i == GRID - 1)
    def _drain():
        for j in range(NBUF):
            pltpu.make_async_copy(
                buf.at[j],
                out_hbm.at[pl.ds(0, R_BLK)],
                sems.at[j],
            ).wait()


def kernel(inputs):
    flat = inputs.reshape(ROWS, IDS_PER_ROW)
    out = pl.pallas_call(
        _body,
        grid=(GRID,),
        in_specs=[pl.BlockSpec((R_BLK, IDS_PER_ROW), lambda i: (i, 0))],
        out_specs=pl.BlockSpec(memory_space=pl.ANY),
        out_shape=jax.ShapeDtypeStruct((ROWS, W), jnp.float32),
        scratch_shapes=[
            pltpu.VMEM((NBUF, R_BLK, W), jnp.float32),
            pltpu.SemaphoreType.DMA((NBUF,)),
        ],
    )(flat)
    return out.reshape(N, S, K)


# speedup vs baseline: 11.8557x; 11.8557x over previous
import jax
import jax.numpy as jnp
from jax.experimental import pallas as pl

N, S, K = 4096, 200, 100
K_BLK = 4
GRID = K // K_BLK


def _body(in_ref, out_ref):
    ids = in_ref[...]
    k0 = pl.program_id(0) * K_BLK
    for kk in range(K_BLK):
        out_ref[kk] = (ids == (k0 + kk)).astype(jnp.float32)


def kernel(inputs):
    x_t = inputs.T
    out_t = pl.pallas_call(
        _body,
        grid=(GRID,),
        in_specs=[pl.BlockSpec((S, N), lambda i: (0, 0))],
        out_specs=pl.BlockSpec((K_BLK, S, N), lambda i: (i, 0, 0)),
        out_shape=jax.ShapeDtypeStruct((K, S, N), jnp.float32),
    )(x_t)
    return jnp.transpose(out_t, (2, 1, 0))
